# baseline (speedup 1.0000x reference)
import functools

import jax
import jax.numpy as jnp
from jax import lax
from jax.experimental import pallas as pl
from jax.experimental.pallas import tpu as pltpu

_S = 64.0
_L_A = 10.0
_U_A = 110.0
_L_MARGIN = 0.45
_U_MARGIN = 0.8

_BR = 8


def _fused_body(V, B, x_ref, emb_ref, lab_ref, o_ref, loss_ref):
    i = pl.program_id(0)

    emb = emb_ref[...]
    xn = jnp.sqrt(jnp.sum(emb * emb, axis=1, keepdims=True))
    xn = jnp.clip(xn, _L_A, _U_A)
    ada = (_U_MARGIN - _L_MARGIN) / (_U_A - _L_A) * (xn - _L_A) + _L_MARGIN
    cos_m = jnp.cos(ada)
    sin_m = jnp.sin(ada)

    g = xn * (1.0 / (_U_A * _U_A)) + 1.0 / xn
    part = jnp.sum(g).reshape(1, 1) / B

    @pl.when(i == 0)
    def _():
        loss_ref[...] = jnp.zeros_like(loss_ref)

    loss_ref[...] += part

    o_ref[...] = x_ref[...] * _S

    v_main = (V // 128) * 128
    tail = V % 128

    def patch(r, c0, width, lab):
        w = x_ref[pl.ds(r, 1), pl.ds(c0, width)]
        m = lax.broadcasted_iota(jnp.int32, (1, width), 1) + c0 == lab
        sin_t = jnp.sqrt(jnp.maximum(1.0 - w * w, 0.0))
        nvw = (w * cos_m[r : r + 1, :] - sin_t * sin_m[r : r + 1, :]) * _S
        o_ref[pl.ds(r, 1), pl.ds(c0, width)] = jnp.where(m, nvw, w * _S)

    for r in range(_BR):
        lab = lab_ref[i * _BR + r]

        @pl.when(lab < v_main)
        def _():
            c0 = pl.multiple_of((lab // 128) * 128, 128)
            patch(r, c0, 128, lab)

        if tail:

            @pl.when(lab >= v_main)
            def _():
                patch(r, v_main, tail, lab)


def kernel(logits, labels, embeddings):
    B, V = logits.shape
    D = embeddings.shape[1]
    labels = labels.astype(jnp.int32)

    out, loss = pl.pallas_call(
        functools.partial(_fused_body, V, B),
        out_shape=(
            jax.ShapeDtypeStruct((B, V), jnp.float32),
            jax.ShapeDtypeStruct((1, 1), jnp.float32),
        ),
        grid=(B // _BR,),
        in_specs=[
            pl.BlockSpec((_BR, V), lambda i: (i, 0)),
            pl.BlockSpec((_BR, D), lambda i: (i, 0)),
            pl.BlockSpec(memory_space=pltpu.SMEM),
        ],
        out_specs=(
            pl.BlockSpec((_BR, V), lambda i: (i, 0)),
            pl.BlockSpec((1, 1), lambda i: (0, 0)),
        ),
    )(logits, embeddings, labels)

    return (out, loss.reshape(()))
